# baseline (speedup 1.0000x reference)
import functools

import jax
import jax.numpy as jnp
from jax import lax
from jax.experimental import pallas as pl
from jax.experimental.pallas import tpu as pltpu
from jax.experimental.pallas import tpu_sc as plsc


def _make_sc_kernel(B, D, b_per_w, num_cores):
    mesh = plsc.VectorSubcoreMesh(core_axis_name="c", subcore_axis_name="s")

    @functools.partial(
        pl.kernel,
        out_type=jax.ShapeDtypeStruct((B,), jnp.float32),
        mesh=mesh,
        compiler_params=pltpu.CompilerParams(needs_layout_passes=False, use_tc_tiling_on_sc=False),
        scratch_types=[
            pltpu.VMEM((b_per_w,), jnp.int32),
            pltpu.VMEM((b_per_w,), jnp.int32),
            pltpu.VMEM((b_per_w, D), jnp.float32),
            pltpu.VMEM((b_per_w, D), jnp.float32),
            pltpu.VMEM((b_per_w,), jnp.float32),
            pltpu.VMEM((b_per_w,), jnp.float32),
            pltpu.SemaphoreType.DMA,
            pltpu.SemaphoreType.DMA,
            pltpu.SemaphoreType.DMA,
        ],
    )
    def sc_kernel(table_hbm, bias_hbm, ida_hbm, idb_hbm, out_hbm,
                  idx_a, idx_b, rows_a, rows_b, bias_v, out_v,
                  sem_a, sem_b, sem_c):
        wid = lax.axis_index("s") * num_cores + lax.axis_index("c")
        base = wid * b_per_w
        pltpu.sync_copy(ida_hbm.at[pl.ds(base, b_per_w)], idx_a)
        pltpu.sync_copy(idb_hbm.at[pl.ds(base, b_per_w)], idx_b)
        ca = pltpu.async_copy(table_hbm.at[idx_a], rows_a, sem_a)
        cb = pltpu.async_copy(table_hbm.at[idx_b], rows_b, sem_b)
        cc = pltpu.async_copy(bias_hbm.at[idx_b], bias_v, sem_c)
        ca.wait()
        cb.wait()
        cc.wait()

        lanes = lax.iota(jnp.int32, 16)

        def body(blk, _):
            row0 = blk * 16
            row_ids = lanes + row0
            acc = bias_v[pl.ds(row0, 16)]
            for d in range(D):
                col = jnp.full((16,), d, jnp.int32)
                va = plsc.load_gather(rows_a, [row_ids, col])
                vb = plsc.load_gather(rows_b, [row_ids, col])
                acc = acc + va * vb
            out_v[pl.ds(row0, 16)] = acc
            return _

        lax.fori_loop(0, b_per_w // 16, body, None)
        pltpu.sync_copy(out_v, out_hbm.at[pl.ds(base, b_per_w)])

    return sc_kernel


def kernel(embedding_matrix, bias, node_id, node_neighbor_id):
    B = node_id.shape[0]
    D = embedding_matrix.shape[1]
    info = plsc.get_sparse_core_info()
    nw = info.num_cores * info.num_subcores
    b_per_w = B // nw
    sc_kernel = _make_sc_kernel(B, D, b_per_w, info.num_cores)
    return sc_kernel(
        embedding_matrix,
        bias,
        node_id.astype(jnp.int32),
        node_neighbor_id.astype(jnp.int32),
    )
